# baseline (speedup 1.0000x reference)
import functools

import jax
import jax.numpy as jnp
from jax import lax
from jax.experimental import pallas as pl
from jax.experimental.pallas import tpu as pltpu
from jax.experimental.pallas import tpu_sc as plsc

_NC, _NS = 2, 16
_NW = _NC * _NS
_HB = 16


def _transpose_out(g3, acc, part, n_parts, full_h):
    Hp, W, BC = g3.shape
    grid = (Hp // _HB, W // 128, BC // 128)
    h_off = part * (Hp // _HB)
    in_specs = [pl.BlockSpec((_HB, 128, 128), lambda hb, wb, cb: (hb, wb, cb))]
    operands = [g3]
    aliases = {}
    if acc is not None:
        in_specs.append(pl.BlockSpec(memory_space=pl.ANY))
        operands.append(acc)
        aliases = {1: 0}

    def body(*refs):
        x_ref, o_ref = refs[0], refs[-1]
        o_ref[...] = jnp.transpose(x_ref[...], (2, 0, 1))

    return pl.pallas_call(
        body,
        grid=grid,
        in_specs=in_specs,
        out_specs=pl.BlockSpec(
            (128, _HB, 128),
            functools.partial(
                lambda off, hb, wb, cb: (cb, off + hb, wb), h_off
            ),
        ),
        out_shape=jax.ShapeDtypeStruct((BC, full_h, W), jnp.float32),
        input_output_aliases=aliases,
        compiler_params=pltpu.CompilerParams(
            dimension_semantics=("parallel", "parallel", "parallel")
        ),
    )(*operands)


def _gather_rows(table, idx, chunk):
    V, D = table.shape
    B = idx.shape[0]
    assert B % (_NW * chunk) == 0
    b_per_w = B // _NW
    n_chunks = b_per_w // chunk
    assert n_chunks % 2 == 0 and n_chunks >= 4
    mesh = plsc.VectorSubcoreMesh(core_axis_name="c", subcore_axis_name="s")

    @functools.partial(
        pl.kernel,
        mesh=mesh,
        out_type=jax.ShapeDtypeStruct((B, D), jnp.float32),
        scratch_types=[
            pltpu.VMEM((b_per_w,), jnp.int32),
            pltpu.VMEM((chunk, D), jnp.float32),
            pltpu.VMEM((chunk, D), jnp.float32),
            pltpu.SemaphoreType.DMA,
            pltpu.SemaphoreType.DMA,
            pltpu.SemaphoreType.DMA,
            pltpu.SemaphoreType.DMA,
        ],
    )
    def k(table_hbm, idx_hbm, out_hbm, idx_v, buf0, buf1, g0, g1, w0, w1):
        wid = lax.axis_index("s") * _NC + lax.axis_index("c")
        base = wid * b_per_w
        pltpu.sync_copy(idx_hbm.at[pl.ds(base, b_per_w)], idx_v)

        def start_g(ci, buf, sem):
            pltpu.async_copy(
                table_hbm.at[idx_v.at[pl.ds(ci * chunk, chunk)]], buf, sem
            )

        def wait_g(buf, sem):
            pltpu.make_async_copy(
                table_hbm.at[idx_v.at[pl.ds(0, chunk)]], buf, sem
            ).wait()

        def start_w(ci, buf, sem):
            pltpu.async_copy(buf, out_hbm.at[pl.ds(base + ci * chunk, chunk)], sem)

        def wait_w(buf, sem):
            pltpu.make_async_copy(buf, out_hbm.at[pl.ds(base, chunk)], sem).wait()

        start_g(0, buf0, g0)
        start_g(1, buf1, g1)

        @pl.loop(0, (n_chunks - 2) // 2)
        def _(k2):
            ci = 2 * k2
            wait_g(buf0, g0)
            start_w(ci, buf0, w0)
            wait_g(buf1, g1)
            start_w(ci + 1, buf1, w1)
            wait_w(buf0, w0)
            start_g(ci + 2, buf0, g0)
            wait_w(buf1, w1)
            start_g(ci + 3, buf1, g1)

        wait_g(buf0, g0)
        start_w(n_chunks - 2, buf0, w0)
        wait_g(buf1, g1)
        start_w(n_chunks - 1, buf1, w1)
        wait_w(buf0, w0)
        wait_w(buf1, w1)

    return k(table, idx)


def kernel(input_features, project_map):
    B, C, H, W = input_features.shape
    Ho, Wo, _ = project_map.shape
    rows = project_map[:, :, 0].astype(jnp.int32)
    cols = project_map[:, :, 1].astype(jnp.int32)
    idx = (rows * W + cols).reshape(-1)

    n_parts = 2
    part_n = (Ho * Wo) // n_parts
    tbl = (
        input_features.reshape(B * C, H, W)
        .transpose(1, 2, 0)
        .reshape(H * W, B * C)
    )
    out3 = None
    for part in range(n_parts):
        idx_p = lax.slice_in_dim(idx, part * part_n, (part + 1) * part_n, axis=0)
        gth = _gather_rows(tbl, idx_p, chunk=128)
        gth3 = gth.reshape(Ho // n_parts, Wo, B * C)
        out3 = _transpose_out(gth3, out3, part, n_parts, Ho)
    return out3.reshape(B, C, Ho, Wo)


# speedup vs baseline: 1.1237x; 1.1237x over previous
import functools

import jax
import jax.numpy as jnp
from jax import lax
from jax.experimental import pallas as pl
from jax.experimental.pallas import tpu as pltpu
from jax.experimental.pallas import tpu_sc as plsc

_NC, _NS = 2, 16
_NW = _NC * _NS


def _gather_rows(table, idx, chunk):
    V, D = table.shape
    B = idx.shape[0]
    assert B % (_NW * chunk) == 0
    b_per_w = B // _NW
    n_chunks = b_per_w // chunk
    mesh = plsc.VectorSubcoreMesh(core_axis_name="c", subcore_axis_name="s")

    assert n_chunks % 2 == 0 and n_chunks >= 4

    @functools.partial(
        pl.kernel,
        mesh=mesh,
        out_type=jax.ShapeDtypeStruct((B, D), jnp.float32),
        scratch_types=[
            pltpu.VMEM((b_per_w,), jnp.int32),
            pltpu.VMEM((chunk, D), jnp.float32),
            pltpu.VMEM((chunk, D), jnp.float32),
            pltpu.SemaphoreType.DMA,
            pltpu.SemaphoreType.DMA,
            pltpu.SemaphoreType.DMA,
            pltpu.SemaphoreType.DMA,
        ],
    )
    def k(table_hbm, idx_hbm, out_hbm, idx_v, buf0, buf1, g0, g1, w0, w1):
        wid = lax.axis_index("s") * _NC + lax.axis_index("c")
        base = wid * b_per_w
        pltpu.sync_copy(idx_hbm.at[pl.ds(base, b_per_w)], idx_v)

        def start_g(ci, buf, sem):
            pltpu.async_copy(
                table_hbm.at[idx_v.at[pl.ds(ci * chunk, chunk)]], buf, sem
            )

        def wait_g(buf, sem):
            pltpu.make_async_copy(
                table_hbm.at[idx_v.at[pl.ds(0, chunk)]], buf, sem
            ).wait()

        def start_w(ci, buf, sem):
            pltpu.async_copy(buf, out_hbm.at[pl.ds(base + ci * chunk, chunk)], sem)

        def wait_w(buf, sem):
            pltpu.make_async_copy(buf, out_hbm.at[pl.ds(base, chunk)], sem).wait()

        start_g(0, buf0, g0)
        start_g(1, buf1, g1)

        @pl.loop(0, (n_chunks - 2) // 2)
        def _(k2):
            ci = 2 * k2
            wait_g(buf0, g0)
            start_w(ci, buf0, w0)
            wait_g(buf1, g1)
            start_w(ci + 1, buf1, w1)
            wait_w(buf0, w0)
            start_g(ci + 2, buf0, g0)
            wait_w(buf1, w1)
            start_g(ci + 3, buf1, g1)

        wait_g(buf0, g0)
        start_w(n_chunks - 2, buf0, w0)
        wait_g(buf1, g1)
        start_w(n_chunks - 1, buf1, w1)
        wait_w(buf0, w0)
        wait_w(buf1, w1)

    return k(table, idx)


def kernel(input_features, project_map):
    B, C, H, W = input_features.shape
    Ho, Wo, _ = project_map.shape
    rows = project_map[:, :, 0].astype(jnp.int32)
    cols = project_map[:, :, 1].astype(jnp.int32)
    idx = (rows * W + cols).reshape(-1)
    table = (
        input_features.reshape(B * C, H, W)
        .transpose(1, 2, 0)
        .reshape(H * W, B * C)
    )
    out_t = _gather_rows(table, idx, chunk=128)
    return out_t.reshape(Ho, Wo, B * C).transpose(2, 0, 1).reshape(B, C, Ho, Wo)


# speedup vs baseline: 1.1252x; 1.0014x over previous
import functools

import jax
import jax.numpy as jnp
from jax import lax
from jax.experimental import pallas as pl
from jax.experimental.pallas import tpu as pltpu
from jax.experimental.pallas import tpu_sc as plsc

_NC, _NS = 2, 16
_NW = _NC * _NS


def _gather_rows(table, idx, chunk):
    V, D = table.shape
    B = idx.shape[0]
    assert B % (_NW * chunk) == 0
    b_per_w = B // _NW
    n_chunks = b_per_w // chunk
    mesh = plsc.VectorSubcoreMesh(core_axis_name="c", subcore_axis_name="s")

    assert n_chunks % 2 == 0 and n_chunks >= 4

    @functools.partial(
        pl.kernel,
        mesh=mesh,
        out_type=jax.ShapeDtypeStruct((B, D), jnp.float32),
        scratch_types=[
            pltpu.VMEM((b_per_w,), jnp.int32),
            pltpu.VMEM((chunk, D), jnp.float32),
            pltpu.VMEM((chunk, D), jnp.float32),
            pltpu.SemaphoreType.DMA,
            pltpu.SemaphoreType.DMA,
            pltpu.SemaphoreType.DMA,
            pltpu.SemaphoreType.DMA,
        ],
    )
    def k(table_hbm, idx_hbm, out_hbm, idx_v, buf0, buf1, g0, g1, w0, w1):
        wid = lax.axis_index("s") * _NC + lax.axis_index("c")
        base = wid * b_per_w
        pltpu.sync_copy(idx_hbm.at[pl.ds(base, b_per_w)], idx_v)

        def start_g(ci, buf, sem):
            pltpu.async_copy(
                table_hbm.at[idx_v.at[pl.ds(ci * chunk, chunk)]], buf, sem
            )

        def wait_g(buf, sem):
            pltpu.make_async_copy(
                table_hbm.at[idx_v.at[pl.ds(0, chunk)]], buf, sem
            ).wait()

        def start_w(ci, buf, sem):
            pltpu.async_copy(buf, out_hbm.at[pl.ds(base + ci * chunk, chunk)], sem)

        def wait_w(buf, sem):
            pltpu.make_async_copy(buf, out_hbm.at[pl.ds(base, chunk)], sem).wait()

        start_g(0, buf0, g0)
        start_g(1, buf1, g1)

        @pl.loop(0, (n_chunks - 2) // 2)
        def _(k2):
            ci = 2 * k2
            wait_g(buf0, g0)
            start_w(ci, buf0, w0)
            wait_g(buf1, g1)
            start_w(ci + 1, buf1, w1)
            wait_w(buf0, w0)
            start_g(ci + 2, buf0, g0)
            wait_w(buf1, w1)
            start_g(ci + 3, buf1, g1)

        wait_g(buf0, g0)
        start_w(n_chunks - 2, buf0, w0)
        wait_g(buf1, g1)
        start_w(n_chunks - 1, buf1, w1)
        wait_w(buf0, w0)
        wait_w(buf1, w1)

    return k(table, idx)


def kernel(input_features, project_map):
    B, C, H, W = input_features.shape
    Ho, Wo, _ = project_map.shape
    rows = project_map[:, :, 0].astype(jnp.int32)
    cols = project_map[:, :, 1].astype(jnp.int32)
    idx = (rows * W + cols).reshape(-1)
    table = (
        input_features.reshape(B * C, H, W)
        .transpose(1, 2, 0)
        .reshape(H * W, B * C)
    )
    out_t = _gather_rows(table, idx, chunk=144)
    return out_t.reshape(Ho, Wo, B * C).transpose(2, 0, 1).reshape(B, C, Ho, Wo)
